# baseline (speedup 1.0000x reference)
import functools

import jax
import jax.numpy as jnp
from jax import lax
from jax.experimental import pallas as pl
from jax.experimental.pallas import tpu as pltpu
from jax.experimental.pallas import tpu_sc as plsc

N = 10000
D = 128
E = 320000
NC, NS = 2, 16
NW = NC * NS
EW = E // NW
C = 125
NK = EW // C
NLIST = 6
SEG = 1000

_mesh = plsc.VectorSubcoreMesh(core_axis_name="c", subcore_axis_name="s")


@functools.partial(
    pl.kernel,
    out_type=jax.ShapeDtypeStruct((NC * 12 * N,), jnp.float32),
    mesh=_mesh,
    scratch_types=[
        pltpu.VMEM((NK, C), jnp.int32),
        pltpu.VMEM((C,), jnp.float32),
        pltpu.VMEM((12 * N // 8,), jnp.float32),
        pltpu.VMEM_SHARED((12 * N,), jnp.float32),
    ],
)
def _deg_kernel(idx_hbm, zeros_hbm, ones_hbm, out_hbm, idx_v, ones_v, bnc_v,
                acc_sh):
    c = lax.axis_index("c")
    s = lax.axis_index("s")
    w = s * NC + c
    pltpu.sync_copy(ones_hbm, ones_v)

    @pl.when(s == 0)
    def _():
        pltpu.sync_copy(zeros_hbm, acc_sh)

    plsc.subcore_barrier()

    def list_body(li, _):
        pltpu.sync_copy(idx_hbm.at[li, w], idx_v)

        def chunk_body(j, _):
            pltpu.sync_copy(ones_v, acc_sh.at[idx_v.at[j]], add=True)
            return 0

        lax.fori_loop(0, NK, chunk_body, 0)
        return 0

    lax.fori_loop(0, 12, list_body, 0)
    plsc.subcore_barrier()

    @pl.when(s < 8)
    def _():
        sz = 12 * N // 8
        pltpu.sync_copy(acc_sh.at[pl.ds(s * sz, sz)], bnc_v)
        pltpu.sync_copy(bnc_v, out_hbm.at[pl.ds(c * (12 * N) + s * sz, sz)])


DH = D // 2
NPASS = 2 * NLIST


@functools.partial(
    pl.kernel,
    out_type=jax.ShapeDtypeStruct((NPASS, NC, N, DH), jnp.float32),
    mesh=_mesh,
    scratch_types=[
        pltpu.VMEM((NK, C), jnp.int32),
        pltpu.VMEM((NK, C), jnp.int32),
        pltpu.VMEM((4, C, DH), jnp.float32),
        pltpu.VMEM((200, DH), jnp.float32),
        pltpu.VMEM_SHARED((N, DH), jnp.float32),
        [pltpu.SemaphoreType.DMA] * 4,
        [pltpu.SemaphoreType.DMA] * 4,
    ],
    compiler_params=pltpu.CompilerParams(use_tc_tiling_on_sc=False),
)
def _agg_kernel(y_hbm, src_hbm, dst_hbm, zeros_hbm, out_hbm,
                src_v, dst_v, rows_v, bnc_v, acc_sh, gsems, ssems):
    c = lax.axis_index("c")
    s = lax.axis_index("s")
    w = s * NC + c

    def gconv_body(g, _):
        @pl.when(s < 10)
        def _():
            pltpu.sync_copy(zeros_hbm.at[pl.ds(s * SEG, SEG)],
                            acc_sh.at[pl.ds(s * SEG, SEG)])

        pltpu.sync_copy(src_hbm.at[g, w], src_v)
        pltpu.sync_copy(dst_hbm.at[g, w], dst_v)
        plsc.subcore_barrier()

        for b in range(2):
            pltpu.async_copy(y_hbm.at[src_v.at[b]], rows_v.at[b], gsems[b])

        def chunk_quad(j0, _):
            for b in range(4):
                j = j0 + b
                pltpu.make_async_copy(y_hbm.at[pl.ds(0, C)], rows_v.at[b],
                                      gsems[b]).wait()
                pltpu.async_copy(rows_v.at[b], acc_sh.at[dst_v.at[j]],
                                 ssems[b], add=True)
                jf = j + 2
                bf = (b + 2) % 4

                @pl.when(jf < NK)
                def _():
                    @pl.when(jf >= 4)
                    def _():
                        pltpu.make_async_copy(y_hbm.at[pl.ds(0, C)],
                                              rows_v.at[bf],
                                              ssems[bf]).wait()

                    pltpu.async_copy(y_hbm.at[src_v.at[jf]], rows_v.at[bf],
                                     gsems[bf])
            return 0

        lax.fori_loop(0, NK // 4, lambda t, u: chunk_quad(4 * t, u), 0)
        for b in range(4):
            pltpu.make_async_copy(y_hbm.at[pl.ds(0, C)], rows_v.at[b],
                                  ssems[b]).wait()
        plsc.subcore_barrier()

        @pl.when(s < 10)
        def _():
            def wb_body(t, _):
                r0 = s * SEG + t * 200
                pltpu.sync_copy(acc_sh.at[pl.ds(r0, 200)], bnc_v)
                pltpu.sync_copy(bnc_v, out_hbm.at[g, c, pl.ds(r0, 200)])
                return 0

            lax.fori_loop(0, 5, wb_body, 0)

        plsc.subcore_barrier()
        return 0

    lax.fori_loop(0, NPASS, gconv_body, 0)


def _norm(deg):
    return jnp.where(deg > 0, lax.rsqrt(jnp.maximum(deg, 1e-12)), 0.0)


def _split_w(w):
    return jnp.stack([w[:, :, :DH], w[:, :, DH:]])


def _mm0_body(x_ref, w_ref, y_ref):
    y_ref[0] = jnp.dot(x_ref[0], w_ref[0, 0],
                       preferred_element_type=jnp.float32)


def _tc_mm0(xstack, w0):
    return pl.pallas_call(
        _mm0_body,
        grid=(NPASS,),
        in_specs=[
            pl.BlockSpec((1, N, D), lambda q: ((q // 2) // 3, 0, 0)),
            pl.BlockSpec((1, 1, D, DH), lambda q: (q % 2, q // 2, 0, 0)),
        ],
        out_specs=pl.BlockSpec((1, N, DH),
                               lambda q: ((q % 2) * 6 + q // 2, 0, 0)),
        out_shape=jax.ShapeDtypeStruct((NPASS, N, DH), jnp.float32),
    )(xstack, _split_w(w0))


def _scale0_body(y_ref, degp_ref, o_ref):
    ns = _norm(degp_ref[0, 0, 0] + degp_ref[1, 0, 0])
    o_ref[0] = y_ref[0] * ns[:, None]


def _tc_scale0(yr, degp):
    return pl.pallas_call(
        _scale0_body,
        grid=(NPASS,),
        in_specs=[
            pl.BlockSpec((1, N, DH), lambda p: (p, 0, 0)),
            pl.BlockSpec((2, 1, 1, N), lambda p: (0, 2 * (p % 6), 0, 0)),
        ],
        out_specs=pl.BlockSpec((1, N, DH), lambda p: (p, 0, 0)),
        out_shape=jax.ShapeDtypeStruct((NPASS, N, DH), jnp.float32),
    )(yr, degp)


def _mid_body(p0_ref, p1_ref, degnd_ref, b0_ref, w1_ref, degns_ref, y_ref):
    nd = _norm(degnd_ref[0, 0, 0] + degnd_ref[1, 0, 0])
    h0 = jnp.maximum((p0_ref[0, 0] + p0_ref[0, 1]) * nd[:, None]
                     + b0_ref[0, :, :DH], 0.0)
    h1 = jnp.maximum((p1_ref[0, 0] + p1_ref[0, 1]) * nd[:, None]
                     + b0_ref[0, :, DH:], 0.0)
    h = jnp.concatenate([h0, h1], axis=1)
    ns = _norm(degns_ref[0, 0, 0] + degns_ref[1, 0, 0])
    y = jnp.dot(h, w1_ref[0, 0], preferred_element_type=jnp.float32)
    y_ref[0] = y * ns[:, None]


def _tc_mid(partials, degp, b0_swapped, w1):
    return pl.pallas_call(
        _mid_body,
        grid=(NPASS,),
        in_specs=[
            pl.BlockSpec((1, 2, N, DH),
                         lambda q: ((q // 2 + 3) % 6, 0, 0, 0)),
            pl.BlockSpec((1, 2, N, DH),
                         lambda q: ((q // 2 + 3) % 6 + 6, 0, 0, 0)),
            pl.BlockSpec((2, 1, 1, N),
                         lambda q: (0, 2 * ((q // 2 + 3) % 6) + 1, 0, 0)),
            pl.BlockSpec((1, 1, D), lambda q: (q // 2, 0, 0)),
            pl.BlockSpec((1, 1, D, DH), lambda q: (q % 2, q // 2, 0, 0)),
            pl.BlockSpec((2, 1, 1, N), lambda q: (0, 2 * (q // 2), 0, 0)),
        ],
        out_specs=pl.BlockSpec((1, N, DH),
                               lambda q: ((q % 2) * 6 + q // 2, 0, 0)),
        out_shape=jax.ShapeDtypeStruct((NPASS, N, DH), jnp.float32),
    )(partials, partials, degp, b0_swapped.reshape(NLIST, 1, D), _split_w(w1),
      degp)


def _final_body(p0_ref, p1_ref, degnd_ref, b1_ref, wa1_ref, ba1_ref,
                wa2_ref, o_ref, wsum_ref):
    nd = _norm(degnd_ref[0, 0, 0] + degnd_ref[1, 0, 0])
    o0 = jnp.maximum((p0_ref[0, 0] + p0_ref[0, 1]) * nd[:, None]
                     + b1_ref[0, :, :DH], 0.0)
    o1 = jnp.maximum((p1_ref[0, 0] + p1_ref[0, 1]) * nd[:, None]
                     + b1_ref[0, :, DH:], 0.0)
    o = jnp.concatenate([o0, o1], axis=1)
    o_ref[0] = o
    a = jnp.tanh(jnp.dot(o, wa1_ref[...], preferred_element_type=jnp.float32)
                 + ba1_ref[0][None, :])
    wv = jnp.sum(a * wa2_ref[0][None, :], axis=1)
    wsum_ref[0] = jnp.full((1, 128), jnp.sum(wv), dtype=jnp.float32)


def _tc_final(partials, degp, b1, wa1, ba1r, wa2r):
    return pl.pallas_call(
        _final_body,
        grid=(NLIST,),
        in_specs=[
            pl.BlockSpec((1, 2, N, DH), lambda g: (g, 0, 0, 0)),
            pl.BlockSpec((1, 2, N, DH), lambda g: (g + 6, 0, 0, 0)),
            pl.BlockSpec((2, 1, 1, N), lambda g: (0, 2 * g + 1, 0, 0)),
            pl.BlockSpec((1, 1, D), lambda g: (g, 0, 0)),
            pl.BlockSpec((D, 16), lambda g: (0, 0)),
            pl.BlockSpec((1, 16), lambda g: (0, 0)),
            pl.BlockSpec((1, 16), lambda g: (0, 0)),
        ],
        out_specs=[
            pl.BlockSpec((1, N, D), lambda g: (g, 0, 0)),
            pl.BlockSpec((1, 1, 128), lambda g: (g, 0, 0)),
        ],
        out_shape=[
            jax.ShapeDtypeStruct((NLIST, N, D), jnp.float32),
            jax.ShapeDtypeStruct((NLIST, 1, 128), jnp.float32),
        ],
    )(partials, partials, degp, b1, wa1, ba1r, wa2r)


def _emb_body(o_ref, wsum_ref, ed_ref, ep_ref, betad_ref, betap_ref):
    wm = wsum_ref[:, 0, 0:1] / float(N)
    beta_p = jax.nn.softmax(wm[0:3], axis=0)
    beta_d = jax.nn.softmax(wm[3:6], axis=0)
    ep_ref[...] = (beta_p[0, 0] * o_ref[0] + beta_p[1, 0] * o_ref[1]
                   + beta_p[2, 0] * o_ref[2])
    ed_ref[...] = (beta_d[0, 0] * o_ref[3] + beta_d[1, 0] * o_ref[4]
                   + beta_d[2, 0] * o_ref[5])
    betad_ref[...] = jnp.pad(beta_d, ((0, 5), (0, 127)))
    betap_ref[...] = jnp.pad(beta_p, ((0, 5), (0, 127)))


def _tc_emb(o, wsum):
    return pl.pallas_call(
        _emb_body,
        out_shape=[
            jax.ShapeDtypeStruct((N, D), jnp.float32),
            jax.ShapeDtypeStruct((N, D), jnp.float32),
            jax.ShapeDtypeStruct((8, 128), jnp.float32),
            jax.ShapeDtypeStruct((8, 128), jnp.float32),
        ],
    )(o, wsum)


def kernel(x_drug, x_protein,
           e0_d2p, e0_p2d, e1_d2p, e1_p2d, e2_d2p, e2_p2d, e3_d2p, e3_p2d,
           W_s0_l0_d2p, b_s0_l0_d2p, W_s0_l0_p2d, b_s0_l0_p2d,
           W_s0_l1_d2p, b_s0_l1_d2p, W_s0_l1_p2d, b_s0_l1_p2d,
           W_s1_l0_d2p, b_s1_l0_d2p, W_s1_l0_p2d, b_s1_l0_p2d,
           W_s1_l1_d2p, b_s1_l1_d2p, W_s1_l1_p2d, b_s1_l1_p2d,
           W_s2_l0_d2p, b_s2_l0_d2p, W_s2_l0_p2d, b_s2_l0_p2d,
           W_s2_l1_d2p, b_s2_l1_d2p, W_s2_l1_p2d, b_s2_l1_p2d,
           W_s3_l0_d2p, b_s3_l0_d2p, W_s3_l0_p2d, b_s3_l0_p2d,
           W_s3_l1_d2p, b_s3_l1_d2p, W_s3_l1_p2d, b_s3_l1_p2d,
           Wa1, ba1, Wa2):
    params = dict(locals())
    edges = ([params[f"e{i}_d2p"] for i in (0, 2, 3)]
             + [params[f"e{i}_p2d"] for i in (0, 2, 3)])
    srcs = [e[0].astype(jnp.int32) for e in edges]
    dsts = [e[1].astype(jnp.int32) for e in edges]

    deg_rows = []
    for g in range(NLIST):
        deg_rows.append(srcs[g] + (2 * g) * N)
        deg_rows.append(dsts[g] + (2 * g + 1) * N)
    deg_idx = jnp.stack(deg_rows).reshape(12, NW, NK, C)
    zeros12 = jnp.zeros((12 * N,), jnp.float32)
    onesC = jnp.ones((C,), jnp.float32)
    degp = _deg_kernel(deg_idx, zeros12, onesC)
    degp = degp.reshape(NC, 12, 1, N)

    src_abs = jnp.stack([srcs[p % NLIST] + p * N for p in range(NPASS)])
    src_abs = src_abs.reshape(NPASS, NW, NK, C)
    dst_idx = jnp.stack([dsts[p % NLIST] for p in range(NPASS)])
    dst_idx = dst_idx.reshape(NPASS, NW, NK, C)
    zerosND = jnp.zeros((N, DH), jnp.float32)

    xstack = jnp.stack([x_drug, x_protein])
    w0 = jnp.stack([params[f"W_s{i}_l0_d2p"] for i in (0, 2, 3)]
                   + [params[f"W_s{i}_l0_p2d"] for i in (0, 2, 3)])
    y0 = _tc_scale0(_tc_mm0(xstack, w0), degp)
    p0 = _agg_kernel(y0.reshape(NPASS * N, DH), src_abs, dst_idx, zerosND)

    b0_swapped = jnp.stack(
        [params[f"b_s{i}_l0_p2d"] for i in (0, 2, 3)]
        + [params[f"b_s{i}_l0_d2p"] for i in (0, 2, 3)])
    w1 = jnp.stack([params[f"W_s{i}_l1_d2p"] for i in (0, 2, 3)]
                   + [params[f"W_s{i}_l1_p2d"] for i in (0, 2, 3)])
    y1 = _tc_mid(p0, degp, b0_swapped, w1)
    p1 = _agg_kernel(y1.reshape(NPASS * N, DH), src_abs, dst_idx, zerosND)

    b1 = jnp.stack([params[f"b_s{i}_l1_d2p"] for i in (0, 2, 3)]
                   + [params[f"b_s{i}_l1_p2d"] for i in (0, 2, 3)])
    o, wsum = _tc_final(p1, degp, b1.reshape(NLIST, 1, D), Wa1, ba1.reshape(1, 16),
                        Wa2.reshape(1, 16))
    ed, ep, betad_pad, betap_pad = _tc_emb(o, wsum)
    return ed, ep, betad_pad[:3, 0:1], betap_pad[:3, 0:1]


# speedup vs baseline: 1.0170x; 1.0170x over previous
import functools

import jax
import jax.numpy as jnp
from jax import lax
from jax.experimental import pallas as pl
from jax.experimental.pallas import tpu as pltpu
from jax.experimental.pallas import tpu_sc as plsc

N = 10000
D = 128
E = 320000
NC, NS = 2, 16
NW = NC * NS
EW = E // NW
C = 125
NK = EW // C
NLIST = 6
SEG = 1000

_mesh = plsc.VectorSubcoreMesh(core_axis_name="c", subcore_axis_name="s")


@functools.partial(
    pl.kernel,
    out_type=jax.ShapeDtypeStruct((NC * 12 * N,), jnp.float32),
    mesh=_mesh,
    scratch_types=[
        pltpu.VMEM((NK, C), jnp.int32),
        pltpu.VMEM((C,), jnp.float32),
        pltpu.VMEM((12 * N // 8,), jnp.float32),
        pltpu.VMEM_SHARED((12 * N,), jnp.float32),
    ],
)
def _deg_kernel(idx_hbm, zeros_hbm, ones_hbm, out_hbm, idx_v, ones_v, bnc_v,
                acc_sh):
    c = lax.axis_index("c")
    s = lax.axis_index("s")
    w = s * NC + c
    pltpu.sync_copy(ones_hbm, ones_v)

    @pl.when(s == 0)
    def _():
        pltpu.sync_copy(zeros_hbm, acc_sh)

    plsc.subcore_barrier()

    def list_body(li, _):
        pltpu.sync_copy(idx_hbm.at[li, w], idx_v)

        def chunk_body(j, _):
            pltpu.sync_copy(ones_v, acc_sh.at[idx_v.at[j]], add=True)
            return 0

        lax.fori_loop(0, NK, chunk_body, 0)
        return 0

    lax.fori_loop(0, 12, list_body, 0)
    plsc.subcore_barrier()

    @pl.when(s < 8)
    def _():
        sz = 12 * N // 8
        pltpu.sync_copy(acc_sh.at[pl.ds(s * sz, sz)], bnc_v)
        pltpu.sync_copy(bnc_v, out_hbm.at[pl.ds(c * (12 * N) + s * sz, sz)])


DH = D // 2
NPASS = 2 * NLIST


@functools.partial(
    pl.kernel,
    out_type=jax.ShapeDtypeStruct((NPASS, NC, N, DH), jnp.float32),
    mesh=_mesh,
    scratch_types=[
        pltpu.VMEM((NK, C), jnp.int32),
        pltpu.VMEM((NK, C), jnp.int32),
        pltpu.VMEM((4, C, DH), jnp.float32),
        pltpu.VMEM((200, DH), jnp.float32),
        pltpu.VMEM_SHARED((N, DH), jnp.float32),
        [pltpu.SemaphoreType.DMA] * 4,
        [pltpu.SemaphoreType.DMA] * 4,
    ],
    compiler_params=pltpu.CompilerParams(use_tc_tiling_on_sc=False),
)
def _agg_kernel(y_hbm, src_hbm, dst_hbm, zeros_hbm, out_hbm,
                src_v, dst_v, rows_v, bnc_v, acc_sh, gsems, ssems):
    c = lax.axis_index("c")
    s = lax.axis_index("s")
    w = s * NC + c

    def gconv_body(g, _):
        @pl.when(s < 10)
        def _():
            pltpu.sync_copy(zeros_hbm.at[pl.ds(s * SEG, SEG)],
                            acc_sh.at[pl.ds(s * SEG, SEG)])

        pltpu.sync_copy(src_hbm.at[g, w], src_v)
        pltpu.sync_copy(dst_hbm.at[g, w], dst_v)
        plsc.subcore_barrier()

        for b in range(2):
            pltpu.async_copy(y_hbm.at[src_v.at[b]], rows_v.at[b], gsems[b])

        def chunk_quad(j0, _):
            for b in range(4):
                j = j0 + b
                pltpu.make_async_copy(y_hbm.at[pl.ds(0, C)], rows_v.at[b],
                                      gsems[b]).wait()
                pltpu.async_copy(rows_v.at[b], acc_sh.at[dst_v.at[j]],
                                 ssems[b], add=True)
                jf = j + 2
                bf = (b + 2) % 4

                @pl.when(jf < NK)
                def _():
                    @pl.when(jf >= 4)
                    def _():
                        pltpu.make_async_copy(y_hbm.at[pl.ds(0, C)],
                                              rows_v.at[bf],
                                              ssems[bf]).wait()

                    pltpu.async_copy(y_hbm.at[src_v.at[jf]], rows_v.at[bf],
                                     gsems[bf])
            return 0

        lax.fori_loop(0, NK // 4, lambda t, u: chunk_quad(4 * t, u), 0)
        for b in range(4):
            pltpu.make_async_copy(y_hbm.at[pl.ds(0, C)], rows_v.at[b],
                                  ssems[b]).wait()
        plsc.subcore_barrier()

        @pl.when(s < 10)
        def _():
            def wb_body(t, _):
                r0 = s * SEG + t * 200
                pltpu.sync_copy(acc_sh.at[pl.ds(r0, 200)], bnc_v)
                pltpu.sync_copy(bnc_v, out_hbm.at[g, c, pl.ds(r0, 200)])
                return 0

            lax.fori_loop(0, 5, wb_body, 0)

        plsc.subcore_barrier()
        return 0

    lax.fori_loop(0, NPASS, gconv_body, 0)


def _norm(deg):
    return jnp.where(deg > 0, lax.rsqrt(jnp.maximum(deg, 1e-12)), 0.0)


def _split_w(w):
    return jnp.stack([w[:, :, :DH], w[:, :, DH:]])


def _mm0_body(x_ref, w_ref, y_ref):
    y_ref[0] = jnp.dot(x_ref[0], w_ref[0, 0],
                       preferred_element_type=jnp.float32)


def _tc_mm0(xstack, w0):
    return pl.pallas_call(
        _mm0_body,
        grid=(NPASS,),
        in_specs=[
            pl.BlockSpec((1, N, D), lambda q: ((q // 2) // 3, 0, 0)),
            pl.BlockSpec((1, 1, D, DH), lambda q: (q % 2, q // 2, 0, 0)),
        ],
        out_specs=pl.BlockSpec((1, N, DH),
                               lambda q: ((q % 2) * 6 + q // 2, 0, 0)),
        out_shape=jax.ShapeDtypeStruct((NPASS, N, DH), jnp.float32),
    )(xstack, _split_w(w0))


def _prep_body(x_ref, w_ref, degp_ref, y_ref):
    ns = _norm(degp_ref[0, 0, 0] + degp_ref[1, 0, 0])
    y = jnp.dot(x_ref[0], w_ref[0, 0], preferred_element_type=jnp.float32)
    y_ref[0] = y * ns[:, None]


def _tc_prep(xstack, w0, degp):
    return pl.pallas_call(
        _prep_body,
        grid=(NPASS,),
        in_specs=[
            pl.BlockSpec((1, N, D), lambda q: ((q // 2) // 3, 0, 0)),
            pl.BlockSpec((1, 1, D, DH), lambda q: (q % 2, q // 2, 0, 0)),
            pl.BlockSpec((2, 1, 1, N), lambda q: (0, 2 * (q // 2), 0, 0)),
        ],
        out_specs=pl.BlockSpec((1, N, DH),
                               lambda q: ((q % 2) * 6 + q // 2, 0, 0)),
        out_shape=jax.ShapeDtypeStruct((NPASS, N, DH), jnp.float32),
    )(xstack, _split_w(w0), degp)


def _mid_body(p0_ref, p1_ref, degnd_ref, b0_ref, w1_ref, degns_ref, y_ref):
    nd = _norm(degnd_ref[0, 0, 0] + degnd_ref[1, 0, 0])
    h0 = jnp.maximum((p0_ref[0, 0] + p0_ref[0, 1]) * nd[:, None]
                     + b0_ref[0, :, :DH], 0.0)
    h1 = jnp.maximum((p1_ref[0, 0] + p1_ref[0, 1]) * nd[:, None]
                     + b0_ref[0, :, DH:], 0.0)
    h = jnp.concatenate([h0, h1], axis=1)
    ns = _norm(degns_ref[0, 0, 0] + degns_ref[1, 0, 0])
    y = jnp.dot(h, w1_ref[0, 0], preferred_element_type=jnp.float32)
    y_ref[0] = y * ns[:, None]


def _tc_mid(partials, degp, b0_swapped, w1):
    return pl.pallas_call(
        _mid_body,
        grid=(NPASS,),
        in_specs=[
            pl.BlockSpec((1, 2, N, DH),
                         lambda q: ((q // 2 + 3) % 6, 0, 0, 0)),
            pl.BlockSpec((1, 2, N, DH),
                         lambda q: ((q // 2 + 3) % 6 + 6, 0, 0, 0)),
            pl.BlockSpec((2, 1, 1, N),
                         lambda q: (0, 2 * ((q // 2 + 3) % 6) + 1, 0, 0)),
            pl.BlockSpec((1, 1, D), lambda q: (q // 2, 0, 0)),
            pl.BlockSpec((1, 1, D, DH), lambda q: (q % 2, q // 2, 0, 0)),
            pl.BlockSpec((2, 1, 1, N), lambda q: (0, 2 * (q // 2), 0, 0)),
        ],
        out_specs=pl.BlockSpec((1, N, DH),
                               lambda q: ((q % 2) * 6 + q // 2, 0, 0)),
        out_shape=jax.ShapeDtypeStruct((NPASS, N, DH), jnp.float32),
    )(partials, partials, degp, b0_swapped.reshape(NLIST, 1, D), _split_w(w1),
      degp)


def _final_body(p0_ref, p1_ref, degnd_ref, b1_ref, wa1_ref, ba1_ref,
                wa2_ref, o_ref, wsum_ref):
    nd = _norm(degnd_ref[0, 0, 0] + degnd_ref[1, 0, 0])
    o0 = jnp.maximum((p0_ref[0, 0] + p0_ref[0, 1]) * nd[:, None]
                     + b1_ref[0, :, :DH], 0.0)
    o1 = jnp.maximum((p1_ref[0, 0] + p1_ref[0, 1]) * nd[:, None]
                     + b1_ref[0, :, DH:], 0.0)
    o = jnp.concatenate([o0, o1], axis=1)
    o_ref[0] = o
    a = jnp.tanh(jnp.dot(o, wa1_ref[...], preferred_element_type=jnp.float32)
                 + ba1_ref[0][None, :])
    wv = jnp.sum(a * wa2_ref[0][None, :], axis=1)
    wsum_ref[0] = jnp.full((1, 128), jnp.sum(wv), dtype=jnp.float32)


def _tc_final(partials, degp, b1, wa1, ba1r, wa2r):
    return pl.pallas_call(
        _final_body,
        grid=(NLIST,),
        in_specs=[
            pl.BlockSpec((1, 2, N, DH), lambda g: (g, 0, 0, 0)),
            pl.BlockSpec((1, 2, N, DH), lambda g: (g + 6, 0, 0, 0)),
            pl.BlockSpec((2, 1, 1, N), lambda g: (0, 2 * g + 1, 0, 0)),
            pl.BlockSpec((1, 1, D), lambda g: (g, 0, 0)),
            pl.BlockSpec((D, 16), lambda g: (0, 0)),
            pl.BlockSpec((1, 16), lambda g: (0, 0)),
            pl.BlockSpec((1, 16), lambda g: (0, 0)),
        ],
        out_specs=[
            pl.BlockSpec((1, N, D), lambda g: (g, 0, 0)),
            pl.BlockSpec((1, 1, 128), lambda g: (g, 0, 0)),
        ],
        out_shape=[
            jax.ShapeDtypeStruct((NLIST, N, D), jnp.float32),
            jax.ShapeDtypeStruct((NLIST, 1, 128), jnp.float32),
        ],
    )(partials, partials, degp, b1, wa1, ba1r, wa2r)


def _emb_body(o_ref, wsum_ref, ed_ref, ep_ref, betad_ref, betap_ref):
    wm = wsum_ref[:, 0, 0:1] / float(N)
    beta_p = jax.nn.softmax(wm[0:3], axis=0)
    beta_d = jax.nn.softmax(wm[3:6], axis=0)
    ep_ref[...] = (beta_p[0, 0] * o_ref[0] + beta_p[1, 0] * o_ref[1]
                   + beta_p[2, 0] * o_ref[2])
    ed_ref[...] = (beta_d[0, 0] * o_ref[3] + beta_d[1, 0] * o_ref[4]
                   + beta_d[2, 0] * o_ref[5])
    betad_ref[...] = jnp.pad(beta_d, ((0, 5), (0, 127)))
    betap_ref[...] = jnp.pad(beta_p, ((0, 5), (0, 127)))


def _tc_emb(o, wsum):
    return pl.pallas_call(
        _emb_body,
        out_shape=[
            jax.ShapeDtypeStruct((N, D), jnp.float32),
            jax.ShapeDtypeStruct((N, D), jnp.float32),
            jax.ShapeDtypeStruct((8, 128), jnp.float32),
            jax.ShapeDtypeStruct((8, 128), jnp.float32),
        ],
    )(o, wsum)


def kernel(x_drug, x_protein,
           e0_d2p, e0_p2d, e1_d2p, e1_p2d, e2_d2p, e2_p2d, e3_d2p, e3_p2d,
           W_s0_l0_d2p, b_s0_l0_d2p, W_s0_l0_p2d, b_s0_l0_p2d,
           W_s0_l1_d2p, b_s0_l1_d2p, W_s0_l1_p2d, b_s0_l1_p2d,
           W_s1_l0_d2p, b_s1_l0_d2p, W_s1_l0_p2d, b_s1_l0_p2d,
           W_s1_l1_d2p, b_s1_l1_d2p, W_s1_l1_p2d, b_s1_l1_p2d,
           W_s2_l0_d2p, b_s2_l0_d2p, W_s2_l0_p2d, b_s2_l0_p2d,
           W_s2_l1_d2p, b_s2_l1_d2p, W_s2_l1_p2d, b_s2_l1_p2d,
           W_s3_l0_d2p, b_s3_l0_d2p, W_s3_l0_p2d, b_s3_l0_p2d,
           W_s3_l1_d2p, b_s3_l1_d2p, W_s3_l1_p2d, b_s3_l1_p2d,
           Wa1, ba1, Wa2):
    params = dict(locals())
    edges = ([params[f"e{i}_d2p"] for i in (0, 2, 3)]
             + [params[f"e{i}_p2d"] for i in (0, 2, 3)])
    srcs = [e[0].astype(jnp.int32) for e in edges]
    dsts = [e[1].astype(jnp.int32) for e in edges]

    deg_rows = []
    for g in range(NLIST):
        deg_rows.append(srcs[g] + (2 * g) * N)
        deg_rows.append(dsts[g] + (2 * g + 1) * N)
    deg_idx = jnp.stack(deg_rows).reshape(12, NW, NK, C)
    zeros12 = jnp.zeros((12 * N,), jnp.float32)
    onesC = jnp.ones((C,), jnp.float32)
    degp = _deg_kernel(deg_idx, zeros12, onesC)
    degp = degp.reshape(NC, 12, 1, N)

    src_abs = jnp.stack([srcs[p % NLIST] + p * N for p in range(NPASS)])
    src_abs = src_abs.reshape(NPASS, NW, NK, C)
    dst_idx = jnp.stack([dsts[p % NLIST] for p in range(NPASS)])
    dst_idx = dst_idx.reshape(NPASS, NW, NK, C)
    zerosND = jnp.zeros((N, DH), jnp.float32)

    xstack = jnp.stack([x_drug, x_protein])
    w0 = jnp.stack([params[f"W_s{i}_l0_d2p"] for i in (0, 2, 3)]
                   + [params[f"W_s{i}_l0_p2d"] for i in (0, 2, 3)])
    y0 = _tc_prep(xstack, w0, degp)
    p0 = _agg_kernel(y0.reshape(NPASS * N, DH), src_abs, dst_idx, zerosND)

    b0_swapped = jnp.stack(
        [params[f"b_s{i}_l0_p2d"] for i in (0, 2, 3)]
        + [params[f"b_s{i}_l0_d2p"] for i in (0, 2, 3)])
    w1 = jnp.stack([params[f"W_s{i}_l1_d2p"] for i in (0, 2, 3)]
                   + [params[f"W_s{i}_l1_p2d"] for i in (0, 2, 3)])
    y1 = _tc_mid(p0, degp, b0_swapped, w1)
    p1 = _agg_kernel(y1.reshape(NPASS * N, DH), src_abs, dst_idx, zerosND)

    b1 = jnp.stack([params[f"b_s{i}_l1_d2p"] for i in (0, 2, 3)]
                   + [params[f"b_s{i}_l1_p2d"] for i in (0, 2, 3)])
    o, wsum = _tc_final(p1, degp, b1.reshape(NLIST, 1, D), Wa1, ba1.reshape(1, 16),
                        Wa2.reshape(1, 16))
    ed, ep, betad_pad, betap_pad = _tc_emb(o, wsum)
    return ed, ep, betad_pad[:3, 0:1], betap_pad[:3, 0:1]
